# Initial kernel scaffold; baseline (speedup 1.0000x reference)
import jax
import jax.numpy as jnp
from jax.experimental import pallas as pl


def kernel(p1, pro1_edge_index, pro1_batch, p2, pro2_edge_index, node_index, W_c1, b_c1, W_c2, b_c2, W_p1fc, b_p1fc, W_p2fc, b_p2fc, W_fc1, b_fc1, W_fc2, b_fc2, W_out, b_out):
    raise NotImplementedError("write your pallas kernel here")


# speedup vs baseline: 15.2820x; 15.2820x over previous
import functools

import jax
import jax.numpy as jnp
from jax import lax
from jax.experimental import pallas as pl
from jax.experimental.pallas import tpu as pltpu
from jax.experimental.pallas import tpu_sc as plsc

N = 10000
E = 160000
D = 256
H = 128
B = 16
NT = 16
CHUNK = 125
NCHUNK = E // NT // CHUNK
ROW_SPLIT = 640
ROW_STAGE = 80
NBLK = 10
BLK = N // NBLK

_mesh = plsc.VectorSubcoreMesh(core_axis_name="c", subcore_axis_name="s")


def _leaky(x):
    return jnp.where(x >= 0, x, 0.01 * x)


@functools.partial(
    pl.kernel,
    mesh=_mesh,
    out_type=[
        jax.ShapeDtypeStruct((N, 16), jnp.float32),
        jax.ShapeDtypeStruct((N, 16), jnp.float32),
    ],
    scratch_types=[
        pltpu.VMEM((NCHUNK, CHUNK), jnp.int32),
        pltpu.VMEM((CHUNK, 16), jnp.float32),
        pltpu.VMEM((ROW_STAGE, 16), jnp.float32),
        pltpu.VMEM_SHARED((N, 16), jnp.float32),
    ],
)
def _sc_degrees(dst1_ref, dst2_ref, out1_ref, out2_ref, idx_v, ones_v, stage_v, acc):
    c = lax.axis_index("c")
    s = lax.axis_index("s")
    base = s * ROW_SPLIT
    nstage = jnp.where(s < NT - 1, ROW_SPLIT // ROW_STAGE,
                       (N - (NT - 1) * ROW_SPLIT) // ROW_STAGE)

    def run(dst_ref, out_ref):
        pltpu.sync_copy(dst_ref.at[s], idx_v)

        def fill_ones(i, _):
            ones_v[i, :] = jnp.ones((16,), jnp.float32)
            return 0
        lax.fori_loop(0, CHUNK, fill_ones, 0)

        def fill_zeros(i, _):
            stage_v[i, :] = jnp.zeros((16,), jnp.float32)
            return 0
        lax.fori_loop(0, ROW_STAGE, fill_zeros, 0)

        def zero(k, _):
            pltpu.sync_copy(stage_v, acc.at[pl.ds(base + k * ROW_STAGE, ROW_STAGE)])
            return 0
        lax.fori_loop(0, nstage, zero, 0)
        plsc.subcore_barrier()

        def scatter(j, _):
            pltpu.sync_copy(ones_v, acc.at[idx_v.at[j]], add=True)
            return 0
        lax.fori_loop(0, NCHUNK, scatter, 0)
        plsc.subcore_barrier()

        def drain(k, _):
            b2 = base + k * ROW_STAGE
            pltpu.sync_copy(acc.at[pl.ds(b2, ROW_STAGE)], stage_v)
            pltpu.sync_copy(stage_v, out_ref.at[pl.ds(b2, ROW_STAGE)])
            return 0
        lax.fori_loop(0, nstage, drain, 0)

    @pl.when(c == 0)
    def _():
        run(dst1_ref, out1_ref)

    @pl.when(c == 1)
    def _():
        run(dst2_ref, out2_ref)


@functools.partial(
    pl.kernel,
    mesh=_mesh,
    out_type=[
        jax.ShapeDtypeStruct((N, H), jnp.float32),
        jax.ShapeDtypeStruct((N, H), jnp.float32),
    ],
    scratch_types=[
        pltpu.VMEM((NCHUNK, CHUNK), jnp.int32),
        pltpu.VMEM((NCHUNK, CHUNK), jnp.int32),
        pltpu.VMEM((CHUNK, H), jnp.float32),
        pltpu.VMEM((ROW_STAGE, H), jnp.float32),
        pltpu.VMEM_SHARED((N, H), jnp.float32),
    ],
)
def _sc_scatter(ha_ref, hb_ref, src_ref, dst_ref, oa_ref, ob_ref,
                src_v, dst_v, rows_v, stage_v, acc):
    c = lax.axis_index("c")
    s = lax.axis_index("s")
    base = s * ROW_SPLIT
    nstage = jnp.where(s < NT - 1, ROW_SPLIT // ROW_STAGE,
                       (N - (NT - 1) * ROW_SPLIT) // ROW_STAGE)

    def run(h_ref, out_ref):
        pltpu.sync_copy(src_ref.at[s], src_v)
        pltpu.sync_copy(dst_ref.at[s], dst_v)

        def init(k, _):
            b2 = base + k * ROW_STAGE
            pltpu.sync_copy(h_ref.at[pl.ds(b2, ROW_STAGE)], stage_v)
            pltpu.sync_copy(stage_v, acc.at[pl.ds(b2, ROW_STAGE)])
            return 0
        lax.fori_loop(0, nstage, init, 0)
        plsc.subcore_barrier()

        def edge_chunk(j, _):
            pltpu.sync_copy(h_ref.at[src_v.at[j]], rows_v)
            pltpu.sync_copy(rows_v, acc.at[dst_v.at[j]], add=True)
            return 0
        lax.fori_loop(0, NCHUNK, edge_chunk, 0)
        plsc.subcore_barrier()

        def drain(k, _):
            b2 = base + k * ROW_STAGE
            pltpu.sync_copy(acc.at[pl.ds(b2, ROW_STAGE)], stage_v)
            pltpu.sync_copy(stage_v, out_ref.at[pl.ds(b2, ROW_STAGE)])
            return 0
        lax.fori_loop(0, nstage, drain, 0)

    @pl.when(c == 0)
    def _():
        run(ha_ref, oa_ref)

    @pl.when(c == 1)
    def _():
        run(hb_ref, ob_ref)


def _tc_prescale_body(p1_ref, p2_ref, d1_ref, d2_ref, w1_ref, w2_ref,
                      h1a_ref, h1b_ref, h2a_ref, h2b_ref):
    def branch(p_ref, d_ref, w_ref, oa_ref, ob_ref):
        deg = 1.0 + d_ref[...][:, 0:1]
        dinv = lax.rsqrt(deg)
        h = jnp.dot(p_ref[...] * dinv, w_ref[...],
                    preferred_element_type=jnp.float32,
                    precision=lax.Precision.HIGHEST)
        oa_ref[...] = h[:, :H]
        ob_ref[...] = h[:, H:]

    branch(p1_ref, d1_ref, w1_ref, h1a_ref, h1b_ref)
    branch(p2_ref, d2_ref, w2_ref, h2a_ref, h2b_ref)


def _tc_prescale(p1, p2, dacc1, dacc2, W_c1, W_c2):
    row = lambda i: (i, 0)
    full = lambda i: (0, 0)
    return pl.pallas_call(
        _tc_prescale_body,
        grid=(NBLK,),
        in_specs=[
            pl.BlockSpec((BLK, D), row),
            pl.BlockSpec((BLK, D), row),
            pl.BlockSpec((BLK, 16), row),
            pl.BlockSpec((BLK, 16), row),
            pl.BlockSpec((D, D), full),
            pl.BlockSpec((D, D), full),
        ],
        out_specs=[pl.BlockSpec((BLK, H), row)] * 4,
        out_shape=[jax.ShapeDtypeStruct((N, H), jnp.float32)] * 4,
    )(p1, p2, dacc1, dacc2, W_c1, W_c2)


def _tc_head_body(s1a_ref, s1b_ref, s2a_ref, s2b_ref, d1_ref, d2_ref,
                  batch_ref, node_ref, bc1_ref, bc2_ref,
                  wp1_ref, bp1_ref, wp2_ref, bp2_ref,
                  wf1_ref, bf1_ref, wf2_ref, bf2_ref, wo_ref, bo_ref,
                  out_ref, seg_sum, seg_cnt, xt_acc):
    i = pl.program_id(0)

    @pl.when(i == 0)
    def _():
        seg_sum[...] = jnp.zeros((B, D), jnp.float32)
        seg_cnt[...] = jnp.zeros((B, D), jnp.float32)
        xt_acc[...] = jnp.zeros((B, D), jnp.float32)

    def post(sa_ref, sb_ref, d_ref, bc_ref):
        svals = jnp.concatenate([sa_ref[...], sb_ref[...]], axis=1)
        deg = 1.0 + d_ref[...][:, 0:1]
        dinv = lax.rsqrt(deg)
        return _leaky(svals * dinv + bc_ref[...])

    x1 = post(s1a_ref, s1b_ref, d1_ref, bc1_ref)
    batch = batch_ref[0, 0, :]
    rows = lax.broadcasted_iota(jnp.int32, (B, BLK), 0)
    oh1 = (rows == jnp.broadcast_to(batch[None, :], (B, BLK))).astype(jnp.float32)
    seg_sum[...] += jnp.dot(oh1, x1, preferred_element_type=jnp.float32,
                            precision=lax.Precision.HIGHEST)
    seg_cnt[...] += jnp.broadcast_to(
        jnp.sum(oh1, axis=1, keepdims=True), (B, D))

    x2 = post(s2a_ref, s2b_ref, d2_ref, bc2_ref)
    colid = lax.broadcasted_iota(jnp.int32, (B, BLK), 1) + i * BLK
    ni = node_ref[...][:, 0:1]
    ohn = (colid == jnp.broadcast_to(ni, (B, BLK))).astype(jnp.float32)
    xt_acc[...] += jnp.dot(ohn, x2, preferred_element_type=jnp.float32,
                           precision=lax.Precision.HIGHEST)

    @pl.when(i == NBLK - 1)
    def _():
        mm = functools.partial(jnp.dot, preferred_element_type=jnp.float32,
                               precision=lax.Precision.HIGHEST)
        xmean = seg_sum[...] / jnp.maximum(seg_cnt[...], 1.0)
        xb1 = _leaky(mm(xmean, wp1_ref[...]) + bp1_ref[...])
        xb2 = _leaky(mm(xt_acc[...], wp2_ref[...]) + bp2_ref[...])
        xc = jnp.concatenate([xb1, xb2], axis=1)
        xc = _leaky(mm(xc, wf1_ref[...]) + bf1_ref[...])
        xc = _leaky(mm(xc, wf2_ref[...]) + bf2_ref[...])
        z = mm(xc, wo_ref[...]) + bo_ref[...]
        out_ref[...] = 1.0 / (1.0 + jnp.exp(-z))


def _tc_head(s1a, s1b, s2a, s2b, dacc1, dacc2, batch3, nodeb,
             bc1, bc2, wp1, bp1, wp2, bp2, wf1, bf1, wf2, bf2, wo, bo):
    row = lambda i: (i, 0)
    full = lambda i: (0, 0)
    fullshape = lambda a: pl.BlockSpec(a.shape, full)
    return pl.pallas_call(
        _tc_head_body,
        grid=(NBLK,),
        in_specs=[
            pl.BlockSpec((BLK, H), row),
            pl.BlockSpec((BLK, H), row),
            pl.BlockSpec((BLK, H), row),
            pl.BlockSpec((BLK, H), row),
            pl.BlockSpec((BLK, 16), row),
            pl.BlockSpec((BLK, 16), row),
            pl.BlockSpec((1, 1, BLK), lambda i: (i, 0, 0)),
            fullshape(nodeb),
            fullshape(bc1), fullshape(bc2),
            fullshape(wp1), fullshape(bp1), fullshape(wp2), fullshape(bp2),
            fullshape(wf1), fullshape(bf1), fullshape(wf2), fullshape(bf2),
            fullshape(wo), fullshape(bo),
        ],
        out_specs=pl.BlockSpec((B, H), full),
        out_shape=jax.ShapeDtypeStruct((B, H), jnp.float32),
        scratch_shapes=[
            pltpu.VMEM((B, D), jnp.float32),
            pltpu.VMEM((B, D), jnp.float32),
            pltpu.VMEM((B, D), jnp.float32),
        ],
    )(s1a, s1b, s2a, s2b, dacc1, dacc2, batch3, nodeb,
      bc1, bc2, wp1, bp1, wp2, bp2, wf1, bf1, wf2, bf2, wo, bo)


def kernel(p1, pro1_edge_index, pro1_batch, p2, pro2_edge_index, node_index,
           W_c1, b_c1, W_c2, b_c2, W_p1fc, b_p1fc, W_p2fc, b_p2fc,
           W_fc1, b_fc1, W_fc2, b_fc2, W_out, b_out):
    src1 = pro1_edge_index[0].reshape(NT, NCHUNK, CHUNK)
    dst1 = pro1_edge_index[1].reshape(NT, NCHUNK, CHUNK)
    src2 = pro2_edge_index[0].reshape(NT, NCHUNK, CHUNK)
    dst2 = pro2_edge_index[1].reshape(NT, NCHUNK, CHUNK)

    dacc1, dacc2 = _sc_degrees(dst1, dst2)
    h1a, h1b, h2a, h2b = _tc_prescale(p1, p2, dacc1, dacc2, W_c1, W_c2)
    s1a, s1b = _sc_scatter(h1a, h1b, src1, dst1)
    s2a, s2b = _sc_scatter(h2a, h2b, src2, dst2)

    batch3 = pro1_batch.reshape(NBLK, 1, BLK)
    nodeb = jnp.broadcast_to(node_index[:, None], (B, H))
    out = _tc_head(
        s1a, s1b, s2a, s2b, dacc1, dacc2, batch3, nodeb,
        b_c1.reshape(1, D), b_c2.reshape(1, D),
        W_p1fc, b_p1fc.reshape(1, H), W_p2fc, b_p2fc.reshape(1, H),
        W_fc1, b_fc1.reshape(1, D), W_fc2, b_fc2.reshape(1, 64),
        jnp.pad(W_out, ((0, 0), (0, H - 1))),
        jnp.pad(b_out.reshape(1, 1), ((0, 0), (0, H - 1))),
    )
    return out[:, :1]
